# baseline (speedup 1.0000x reference)
import functools

import jax
import jax.numpy as jnp
from jax import lax
from jax.experimental import pallas as pl
from jax.experimental.pallas import tpu as pltpu
from jax.experimental.pallas import tpu_sc as plsc

_NC = 2
_NS = 16
_NW = _NC * _NS
_B = 128
_NBUF = 6
_LOOK = 3


@functools.lru_cache(maxsize=None)
def _edge_agg(n: int, e: int, h: int):
    epw = e // _NW
    nblk = epw // _B
    tail = epw - nblk * _B
    rchunk = 80
    nchunks = n // rchunk
    assert nchunks * rchunk == n
    chunks_per_tile = -(-nchunks // _NS)

    mesh = plsc.VectorSubcoreMesh(core_axis_name="c", subcore_axis_name="s")

    @functools.partial(
        pl.kernel,
        out_type=jax.ShapeDtypeStruct((_NC, n, h), jnp.float32),
        mesh=mesh,
        compiler_params=pltpu.CompilerParams(
            needs_layout_passes=False, use_tc_tiling_on_sc=False),
        scratch_types=[
            pltpu.VMEM((epw,), jnp.int32),
            pltpu.VMEM((nblk, _B), jnp.int32),
            pltpu.VMEM((max(tail, 8),), jnp.int32),
            pltpu.VMEM((epw,), jnp.float32),
            [pltpu.VMEM((_B, h), jnp.float32)] * _NBUF,
            pltpu.VMEM((rchunk, h), jnp.float32),
            pltpu.VMEM_SHARED((n, h), jnp.float32),
            [pltpu.SemaphoreType.DMA] * _NBUF,
            [pltpu.SemaphoreType.DMA] * _NBUF,
        ],
    )
    def agg_kernel(h_hbm, src_hbm, dst_hbm, dstt_hbm, ew_hbm, out_hbm,
                   src_v, dst_v, dstt_v, ew_v, bufs, zero_v, agg_sh,
                   gsems, ssems):
        cid = lax.axis_index("c")
        sid = lax.axis_index("s")
        wid = cid * _NS + sid

        pltpu.sync_copy(src_hbm.at[wid], src_v)
        pltpu.sync_copy(dst_hbm.at[wid], dst_v)
        if tail:
            pltpu.sync_copy(dstt_hbm.at[wid], dstt_v)
        pltpu.sync_copy(ew_hbm.at[wid], ew_v)

        z16 = jnp.zeros((16,), jnp.float32)

        def zero_body(i, _):
            zero_v[i, pl.ds(0, 16)] = z16
            zero_v[i, pl.ds(16, 16)] = z16
            return 0

        lax.fori_loop(0, rchunk, zero_body, 0)
        for k in range(chunks_per_tile):
            c = sid + _NS * k

            @pl.when(c < nchunks)
            def _():
                pltpu.sync_copy(zero_v, agg_sh.at[pl.ds(c * rchunk, rchunk)])

        plsc.subcore_barrier()

        def gather_desc(jj, p):
            return pltpu.make_async_copy(
                h_hbm.at[src_v.at[pl.ds(jj * _B, _B)]], bufs[p], gsems[p])

        def scatter_desc(jj, p):
            return pltpu.make_async_copy(
                bufs[p], agg_sh.at[dst_v.at[jj]], ssems[p])

        def scale_rows(cur, j, nrows):
            def group_body(g, _):
                w16 = ew_v[pl.ds(j * _B + g * 16, 16)]
                for ell in range(16):
                    wl = w16.at[jnp.full((16,), ell, jnp.int32)].get(
                        mode="promise_in_bounds")
                    i = g * 16 + ell
                    cur[i, pl.ds(0, 16)] = cur[i, pl.ds(0, 16)] * wl
                    cur[i, pl.ds(16, 16)] = cur[i, pl.ds(16, 16)] * wl
                return 0

            lax.fori_loop(0, nrows // 16, group_body, 0, unroll=2)

        for jj in range(min(_LOOK, nblk)):
            pltpu.async_copy(
                h_hbm.at[src_v.at[pl.ds(jj * _B, _B)]], bufs[jj], gsems[jj])

        def blk_body(j, _):
            def phase(p):
                r = (p + _LOOK) % _NBUF
                gather_desc(j, p).wait()

                @pl.when(j + _LOOK < nblk)
                def _():
                    @pl.when(j >= _LOOK)
                    def _():
                        scatter_desc(j - _LOOK, r).wait()

                    idx = src_v.at[pl.ds((j + _LOOK) * _B, _B)]
                    pltpu.async_copy(h_hbm.at[idx], bufs[r], gsems[r])

                scale_rows(bufs[p], j, _B)
                pltpu.async_copy(bufs[p], agg_sh.at[dst_v.at[j]], ssems[p],
                                 add=True)

            for p in range(_NBUF):
                @pl.when(j % _NBUF == p)
                def _(p=p):
                    phase(p)

            return 0

        lax.fori_loop(0, nblk, blk_body, 0)
        for jj in range(max(nblk - _LOOK, 0), nblk):
            scatter_desc(jj, jj % _NBUF).wait()

        if tail:
            idx = src_v.at[pl.ds(nblk * _B, tail)]
            tbuf = bufs[0].at[pl.ds(0, tail)]
            pltpu.async_copy(h_hbm.at[idx], tbuf, gsems[0]).wait()
            scale_rows(bufs[0], nblk, tail)
            pltpu.sync_copy(tbuf, agg_sh.at[dstt_v], add=True)

        plsc.subcore_barrier()
        for k in range(chunks_per_tile):
            c = sid + _NS * k

            @pl.when(c < nchunks)
            def _():
                pltpu.sync_copy(agg_sh.at[pl.ds(c * rchunk, rchunk)],
                                out_hbm.at[cid, pl.ds(c * rchunk, rchunk)])

    return agg_kernel


def _tc_call(body, out_shapes, *args):
    return pl.pallas_call(
        body,
        out_shape=[jax.ShapeDtypeStruct(s, jnp.float32) for s in out_shapes],
    )(*args)


def _mm2_body(x_ref, wn_ref, ws_ref, hm_ref, sm_ref):
    x = x_ref[...]
    hm_ref[...] = jnp.dot(x, wn_ref[...], preferred_element_type=jnp.float32)
    sm_ref[...] = jnp.dot(x, ws_ref[...], preferred_element_type=jnp.float32)


def _combine_body(agg_ref, s_ref, b_ref, wn_ref, ws_ref, hm_ref, sm_ref):
    hcur = jax.nn.relu(agg_ref[0] + agg_ref[1] + b_ref[...] + s_ref[...])
    hm_ref[...] = jnp.dot(hcur, wn_ref[...], preferred_element_type=jnp.float32)
    sm_ref[...] = jnp.dot(hcur, ws_ref[...], preferred_element_type=jnp.float32)


def _final_body(agg_ref, s_ref, b_ref, wfc1_ref, bfc1_ref, wout_ref, bout_ref,
                out_ref):
    hcur = jax.nn.relu(agg_ref[0] + agg_ref[1] + b_ref[...] + s_ref[...])
    hg = jnp.sum(hcur, axis=0, keepdims=True)
    hg2 = jax.nn.relu(
        jnp.dot(hg, wfc1_ref[...], preferred_element_type=jnp.float32)
        + bfc1_ref[...])
    o = jax.nn.relu(
        jnp.dot(hg2, wout_ref[...], preferred_element_type=jnp.float32)
        + bout_ref[...])
    out_ref[...] = jax.nn.softmax(o, axis=1)


def kernel(x, edge_index, edge_weight, W_neigh0, W_self0, b_neigh0,
           W_neigh_h, W_self_h, b_neigh_h, W_fc1, b_fc1, W_out, b_out):
    n, d = x.shape
    e = edge_index.shape[1]
    h = W_neigh0.shape[1]
    epw = e // _NW

    nblk = epw // _B
    src_r = edge_index[0].reshape(_NW, epw)
    dst_flat = edge_index[1].reshape(_NW, epw)
    dst_r = dst_flat[:, :nblk * _B].reshape(_NW, nblk, _B)
    dst_t = dst_flat[:, nblk * _B:]
    if dst_t.shape[1] == 0:
        dst_t = jnp.zeros((_NW, 8), jnp.int32)
    ew_r = edge_weight.reshape(_NW, epw)

    agg_fn = _edge_agg(n, e, h)

    hm, sm = _tc_call(_mm2_body, [(n, h), (n, h)], x, W_neigh0, W_self0)

    biases = [b_neigh0.reshape(1, h)] + [b_neigh_h[i].reshape(1, h)
                                         for i in range(3)]
    for i in range(3):
        agg = agg_fn(hm, src_r, dst_r, dst_t, ew_r)
        hm, sm = _tc_call(_combine_body, [(n, h), (n, h)],
                          agg, sm, biases[i], W_neigh_h[i], W_self_h[i])

    agg = agg_fn(hm, src_r, dst_r, dst_t, ew_r)
    (out,) = _tc_call(_final_body, [(1, 4)],
                      agg, sm, biases[3], W_fc1, b_fc1.reshape(1, 8),
                      W_out, b_out.reshape(1, 4))
    return out


# speedup vs baseline: 1.2193x; 1.2193x over previous
import functools

import jax
import jax.numpy as jnp
from jax import lax
from jax.experimental import pallas as pl
from jax.experimental.pallas import tpu as pltpu
from jax.experimental.pallas import tpu_sc as plsc

_NC = 2
_NS = 16
_NW = _NC * _NS
_B = 128
_NBUF = 6
_LOOK = 3


@functools.lru_cache(maxsize=None)
def _edge_agg(n: int, e: int, h: int):
    epw = e // _NW
    nblk = epw // _B
    tail = epw - nblk * _B
    rchunk = 80
    nchunks = n // rchunk
    assert nchunks * rchunk == n
    chunks_per_tile = -(-nchunks // _NS)

    mesh = plsc.VectorSubcoreMesh(core_axis_name="c", subcore_axis_name="s")

    @functools.partial(
        pl.kernel,
        out_type=jax.ShapeDtypeStruct((_NC, n, h), jnp.float32),
        mesh=mesh,
        compiler_params=pltpu.CompilerParams(
            needs_layout_passes=False, use_tc_tiling_on_sc=False),
        scratch_types=[
            pltpu.VMEM((epw,), jnp.int32),
            pltpu.VMEM((nblk, _B), jnp.int32),
            pltpu.VMEM((max(tail, 8),), jnp.int32),
            pltpu.VMEM((epw,), jnp.float32),
            [pltpu.VMEM((_B, h), jnp.float32)] * _NBUF,
            pltpu.VMEM((rchunk, h), jnp.float32),
            pltpu.VMEM_SHARED((n, h), jnp.float32),
            [pltpu.SemaphoreType.DMA] * _NBUF,
            [pltpu.SemaphoreType.DMA] * _NBUF,
        ],
    )
    def agg_kernel(h_hbm, src_hbm, dst_hbm, dstt_hbm, ew_hbm, out_hbm,
                   src_v, dst_v, dstt_v, ew_v, bufs, zero_v, agg_sh,
                   gsems, ssems):
        cid = lax.axis_index("c")
        sid = lax.axis_index("s")
        wid = cid * _NS + sid

        pltpu.sync_copy(src_hbm.at[wid], src_v)
        pltpu.sync_copy(dst_hbm.at[wid], dst_v)
        if tail:
            pltpu.sync_copy(dstt_hbm.at[wid], dstt_v)
        pltpu.sync_copy(ew_hbm.at[wid], ew_v)

        z16 = jnp.zeros((16,), jnp.float32)

        def zero_body(i, _):
            zero_v[i, pl.ds(0, 16)] = z16
            zero_v[i, pl.ds(16, 16)] = z16
            return 0

        lax.fori_loop(0, rchunk, zero_body, 0)
        for k in range(chunks_per_tile):
            c = sid + _NS * k

            @pl.when(c < nchunks)
            def _():
                pltpu.sync_copy(zero_v, agg_sh.at[pl.ds(c * rchunk, rchunk)])

        plsc.subcore_barrier()

        def gather_desc(jj, p):
            return pltpu.make_async_copy(
                h_hbm.at[src_v.at[pl.ds(jj * _B, _B)]], bufs[p], gsems[p])

        def scatter_desc(jj, p):
            return pltpu.make_async_copy(
                bufs[p], agg_sh.at[dst_v.at[jj]], ssems[p])

        def scale_rows(cur, j, nrows):
            def scale_body(i, _):
                wsplat = plsc.load_gather(
                    ew_v, [jnp.full((16,), j * _B + i, jnp.int32)])
                cur[i, pl.ds(0, 16)] = cur[i, pl.ds(0, 16)] * wsplat
                cur[i, pl.ds(16, 16)] = cur[i, pl.ds(16, 16)] * wsplat
                return 0

            lax.fori_loop(0, nrows, scale_body, 0, unroll=8)

        for jj in range(min(_LOOK, nblk)):
            pltpu.async_copy(
                h_hbm.at[src_v.at[pl.ds(jj * _B, _B)]], bufs[jj], gsems[jj])

        def blk_body(j, _):
            def phase(p):
                r = (p + _LOOK) % _NBUF
                gather_desc(j, p).wait()

                @pl.when(j + _LOOK < nblk)
                def _():
                    @pl.when(j >= _LOOK)
                    def _():
                        scatter_desc(j - _LOOK, r).wait()

                    idx = src_v.at[pl.ds((j + _LOOK) * _B, _B)]
                    pltpu.async_copy(h_hbm.at[idx], bufs[r], gsems[r])

                scale_rows(bufs[p], j, _B)
                pltpu.async_copy(bufs[p], agg_sh.at[dst_v.at[j]], ssems[p],
                                 add=True)

            for p in range(_NBUF):
                @pl.when(j % _NBUF == p)
                def _(p=p):
                    phase(p)

            return 0

        lax.fori_loop(0, nblk, blk_body, 0)
        for jj in range(max(nblk - _LOOK, 0), nblk):
            scatter_desc(jj, jj % _NBUF).wait()

        if tail:
            idx = src_v.at[pl.ds(nblk * _B, tail)]
            tbuf = bufs[0].at[pl.ds(0, tail)]
            pltpu.async_copy(h_hbm.at[idx], tbuf, gsems[0]).wait()
            scale_rows(bufs[0], nblk, tail)
            pltpu.sync_copy(tbuf, agg_sh.at[dstt_v], add=True)

        plsc.subcore_barrier()
        for k in range(chunks_per_tile):
            c = sid + _NS * k

            @pl.when(c < nchunks)
            def _():
                pltpu.sync_copy(agg_sh.at[pl.ds(c * rchunk, rchunk)],
                                out_hbm.at[cid, pl.ds(c * rchunk, rchunk)])

    return agg_kernel


def _tc_call(body, out_shapes, *args):
    return pl.pallas_call(
        body,
        out_shape=[jax.ShapeDtypeStruct(s, jnp.float32) for s in out_shapes],
    )(*args)


def _mm2_body(x_ref, wn_ref, ws_ref, hm_ref, sm_ref):
    x = x_ref[...]
    hm_ref[...] = jnp.dot(x, wn_ref[...], preferred_element_type=jnp.float32)
    sm_ref[...] = jnp.dot(x, ws_ref[...], preferred_element_type=jnp.float32)


def _combine_body(agg_ref, s_ref, b_ref, wn_ref, ws_ref, hm_ref, sm_ref):
    hcur = jax.nn.relu(agg_ref[0] + agg_ref[1] + b_ref[...] + s_ref[...])
    hm_ref[...] = jnp.dot(hcur, wn_ref[...], preferred_element_type=jnp.float32)
    sm_ref[...] = jnp.dot(hcur, ws_ref[...], preferred_element_type=jnp.float32)


def _final_body(agg_ref, s_ref, b_ref, wfc1_ref, bfc1_ref, wout_ref, bout_ref,
                out_ref):
    hcur = jax.nn.relu(agg_ref[0] + agg_ref[1] + b_ref[...] + s_ref[...])
    hg = jnp.sum(hcur, axis=0, keepdims=True)
    hg2 = jax.nn.relu(
        jnp.dot(hg, wfc1_ref[...], preferred_element_type=jnp.float32)
        + bfc1_ref[...])
    o = jax.nn.relu(
        jnp.dot(hg2, wout_ref[...], preferred_element_type=jnp.float32)
        + bout_ref[...])
    out_ref[...] = jax.nn.softmax(o, axis=1)


def kernel(x, edge_index, edge_weight, W_neigh0, W_self0, b_neigh0,
           W_neigh_h, W_self_h, b_neigh_h, W_fc1, b_fc1, W_out, b_out):
    n, d = x.shape
    e = edge_index.shape[1]
    h = W_neigh0.shape[1]
    epw = e // _NW

    nblk = epw // _B
    src_r = edge_index[0].reshape(_NW, epw)
    dst_flat = edge_index[1].reshape(_NW, epw)
    dst_r = dst_flat[:, :nblk * _B].reshape(_NW, nblk, _B)
    dst_t = dst_flat[:, nblk * _B:]
    if dst_t.shape[1] == 0:
        dst_t = jnp.zeros((_NW, 8), jnp.int32)
    ew_r = edge_weight.reshape(_NW, epw)

    agg_fn = _edge_agg(n, e, h)

    hm, sm = _tc_call(_mm2_body, [(n, h), (n, h)], x, W_neigh0, W_self0)

    biases = [b_neigh0.reshape(1, h)] + [b_neigh_h[i].reshape(1, h)
                                         for i in range(3)]
    for i in range(3):
        agg = agg_fn(hm, src_r, dst_r, dst_t, ew_r)
        hm, sm = _tc_call(_combine_body, [(n, h), (n, h)],
                          agg, sm, biases[i], W_neigh_h[i], W_self_h[i])

    agg = agg_fn(hm, src_r, dst_r, dst_t, ew_r)
    (out,) = _tc_call(_final_body, [(1, 4)],
                      agg, sm, biases[3], W_fc1, b_fc1.reshape(1, 8),
                      W_out, b_out.reshape(1, 4))
    return out


# speedup vs baseline: 1.2333x; 1.0115x over previous
import functools

import jax
import jax.numpy as jnp
from jax import lax
from jax.experimental import pallas as pl
from jax.experimental.pallas import tpu as pltpu
from jax.experimental.pallas import tpu_sc as plsc

_NC = 2
_NS = 16
_NW = _NC * _NS
_B = 400
_NBUF = 6
_LOOK = 3


@functools.lru_cache(maxsize=None)
def _edge_agg(n: int, e: int, h: int):
    epw = e // _NW
    nblk = epw // _B
    tail = epw - nblk * _B
    rchunk = 80
    nchunks = n // rchunk
    assert nchunks * rchunk == n
    chunks_per_tile = -(-nchunks // _NS)

    mesh = plsc.VectorSubcoreMesh(core_axis_name="c", subcore_axis_name="s")

    @functools.partial(
        pl.kernel,
        out_type=jax.ShapeDtypeStruct((_NC, n, h), jnp.float32),
        mesh=mesh,
        compiler_params=pltpu.CompilerParams(
            needs_layout_passes=False, use_tc_tiling_on_sc=False),
        scratch_types=[
            pltpu.VMEM((epw,), jnp.int32),
            pltpu.VMEM((nblk, _B), jnp.int32),
            pltpu.VMEM((max(tail, 8),), jnp.int32),
            pltpu.VMEM((epw,), jnp.float32),
            [pltpu.VMEM((_B, h), jnp.float32)] * _NBUF,
            pltpu.VMEM((rchunk, h), jnp.float32),
            pltpu.VMEM_SHARED((n, h), jnp.float32),
            [pltpu.SemaphoreType.DMA] * _NBUF,
            [pltpu.SemaphoreType.DMA] * _NBUF,
        ],
    )
    def agg_kernel(h_hbm, src_hbm, dst_hbm, dstt_hbm, ew_hbm, out_hbm,
                   src_v, dst_v, dstt_v, ew_v, bufs, zero_v, agg_sh,
                   gsems, ssems):
        cid = lax.axis_index("c")
        sid = lax.axis_index("s")
        wid = cid * _NS + sid

        pltpu.sync_copy(src_hbm.at[wid], src_v)
        pltpu.sync_copy(dst_hbm.at[wid], dst_v)
        if tail:
            pltpu.sync_copy(dstt_hbm.at[wid], dstt_v)
        pltpu.sync_copy(ew_hbm.at[wid], ew_v)

        z16 = jnp.zeros((16,), jnp.float32)

        def zero_body(i, _):
            zero_v[i, pl.ds(0, 16)] = z16
            zero_v[i, pl.ds(16, 16)] = z16
            return 0

        lax.fori_loop(0, rchunk, zero_body, 0)
        for k in range(chunks_per_tile):
            c = sid + _NS * k

            @pl.when(c < nchunks)
            def _():
                pltpu.sync_copy(zero_v, agg_sh.at[pl.ds(c * rchunk, rchunk)])

        plsc.subcore_barrier()

        def gather_desc(jj, p):
            return pltpu.make_async_copy(
                h_hbm.at[src_v.at[pl.ds(jj * _B, _B)]], bufs[p], gsems[p])

        def scatter_desc(jj, p):
            return pltpu.make_async_copy(
                bufs[p], agg_sh.at[dst_v.at[jj]], ssems[p])

        def scale_rows(cur, j, nrows):
            def scale_body(i, _):
                wsplat = plsc.load_gather(
                    ew_v, [jnp.full((16,), j * _B + i, jnp.int32)])
                cur[i, pl.ds(0, 16)] = cur[i, pl.ds(0, 16)] * wsplat
                cur[i, pl.ds(16, 16)] = cur[i, pl.ds(16, 16)] * wsplat
                return 0

            lax.fori_loop(0, nrows, scale_body, 0, unroll=8)

        for jj in range(min(_LOOK, nblk)):
            pltpu.async_copy(
                h_hbm.at[src_v.at[pl.ds(jj * _B, _B)]], bufs[jj], gsems[jj])

        def blk_body(j, _):
            def phase(p):
                r = (p + _LOOK) % _NBUF
                gather_desc(j, p).wait()

                @pl.when(j + _LOOK < nblk)
                def _():
                    @pl.when(j >= _LOOK)
                    def _():
                        scatter_desc(j - _LOOK, r).wait()

                    idx = src_v.at[pl.ds((j + _LOOK) * _B, _B)]
                    pltpu.async_copy(h_hbm.at[idx], bufs[r], gsems[r])

                scale_rows(bufs[p], j, _B)
                pltpu.async_copy(bufs[p], agg_sh.at[dst_v.at[j]], ssems[p],
                                 add=True)

            for p in range(_NBUF):
                @pl.when(j % _NBUF == p)
                def _(p=p):
                    phase(p)

            return 0

        lax.fori_loop(0, nblk, blk_body, 0)
        for jj in range(max(nblk - _LOOK, 0), nblk):
            scatter_desc(jj, jj % _NBUF).wait()

        if tail:
            idx = src_v.at[pl.ds(nblk * _B, tail)]
            tbuf = bufs[0].at[pl.ds(0, tail)]
            pltpu.async_copy(h_hbm.at[idx], tbuf, gsems[0]).wait()
            scale_rows(bufs[0], nblk, tail)
            pltpu.sync_copy(tbuf, agg_sh.at[dstt_v], add=True)

        plsc.subcore_barrier()
        for k in range(chunks_per_tile):
            c = sid + _NS * k

            @pl.when(c < nchunks)
            def _():
                pltpu.sync_copy(agg_sh.at[pl.ds(c * rchunk, rchunk)],
                                out_hbm.at[cid, pl.ds(c * rchunk, rchunk)])

    return agg_kernel


def _tc_call(body, out_shapes, *args):
    return pl.pallas_call(
        body,
        out_shape=[jax.ShapeDtypeStruct(s, jnp.float32) for s in out_shapes],
    )(*args)


def _mm2_body(x_ref, wn_ref, ws_ref, hm_ref, sm_ref):
    x = x_ref[...]
    hm_ref[...] = jnp.dot(x, wn_ref[...], preferred_element_type=jnp.float32)
    sm_ref[...] = jnp.dot(x, ws_ref[...], preferred_element_type=jnp.float32)


def _combine_body(agg_ref, s_ref, b_ref, wn_ref, ws_ref, hm_ref, sm_ref):
    hcur = jax.nn.relu(agg_ref[0] + agg_ref[1] + b_ref[...] + s_ref[...])
    hm_ref[...] = jnp.dot(hcur, wn_ref[...], preferred_element_type=jnp.float32)
    sm_ref[...] = jnp.dot(hcur, ws_ref[...], preferred_element_type=jnp.float32)


def _final_body(agg_ref, s_ref, b_ref, wfc1_ref, bfc1_ref, wout_ref, bout_ref,
                out_ref):
    hcur = jax.nn.relu(agg_ref[0] + agg_ref[1] + b_ref[...] + s_ref[...])
    hg = jnp.sum(hcur, axis=0, keepdims=True)
    hg2 = jax.nn.relu(
        jnp.dot(hg, wfc1_ref[...], preferred_element_type=jnp.float32)
        + bfc1_ref[...])
    o = jax.nn.relu(
        jnp.dot(hg2, wout_ref[...], preferred_element_type=jnp.float32)
        + bout_ref[...])
    out_ref[...] = jax.nn.softmax(o, axis=1)


def kernel(x, edge_index, edge_weight, W_neigh0, W_self0, b_neigh0,
           W_neigh_h, W_self_h, b_neigh_h, W_fc1, b_fc1, W_out, b_out):
    n, d = x.shape
    e = edge_index.shape[1]
    h = W_neigh0.shape[1]
    epw = e // _NW

    nblk = epw // _B
    src_r = edge_index[0].reshape(_NW, epw)
    dst_flat = edge_index[1].reshape(_NW, epw)
    dst_r = dst_flat[:, :nblk * _B].reshape(_NW, nblk, _B)
    dst_t = dst_flat[:, nblk * _B:]
    if dst_t.shape[1] == 0:
        dst_t = jnp.zeros((_NW, 8), jnp.int32)
    ew_r = edge_weight.reshape(_NW, epw)

    agg_fn = _edge_agg(n, e, h)

    hm, sm = _tc_call(_mm2_body, [(n, h), (n, h)], x, W_neigh0, W_self0)

    biases = [b_neigh0.reshape(1, h)] + [b_neigh_h[i].reshape(1, h)
                                         for i in range(3)]
    for i in range(3):
        agg = agg_fn(hm, src_r, dst_r, dst_t, ew_r)
        hm, sm = _tc_call(_combine_body, [(n, h), (n, h)],
                          agg, sm, biases[i], W_neigh_h[i], W_self_h[i])

    agg = agg_fn(hm, src_r, dst_r, dst_t, ew_r)
    (out,) = _tc_call(_final_body, [(1, 4)],
                      agg, sm, biases[3], W_fc1, b_fc1.reshape(1, 8),
                      W_out, b_out.reshape(1, 4))
    return out
